# baseline (speedup 1.0000x reference)
import functools

import jax
import jax.numpy as jnp
from jax import lax
from jax.experimental import pallas as pl
from jax.experimental.pallas import tpu as pltpu
from jax.experimental.pallas import tpu_sc as plsc

N_USERS = 50000
N_NODES = 100000
DIM = 32
HDIM = DIM // 2
N_LAYERS = 3

NC = 2
NS = 16
LANES = 16
CHUNK = 128
G = 6
G_DEG = 3

ACC2_ROWS = N_NODES + 8
DRAIN = 400
Z_STRIPE = 6400
Z_LAST = N_NODES - (NS - 1) * Z_STRIPE
PREP = 800

_MESH = plsc.VectorSubcoreMesh(core_axis_name="c", subcore_axis_name="s")
_SC_PARAMS = pltpu.CompilerParams(use_tc_tiling_on_sc=False)
_SC_PARAMS_NL = pltpu.CompilerParams(use_tc_tiling_on_sc=False,
                                     needs_layout_passes=False)


def _n_stripe_chunks(tile, chunk_rows):
    return jnp.where(tile < NS - 1, Z_STRIPE // chunk_rows,
                     Z_LAST // chunk_rows)


def _sc_degree(col2):
    total_chunks = col2.shape[0]
    per_tile = total_chunks // (NC * NS)
    n_groups = per_tile // G_DEG

    @functools.partial(
        pl.kernel,
        out_type=jax.ShapeDtypeStruct((NC, N_NODES, HDIM), jnp.float32),
        mesh=_MESH,
        scratch_types=[
            pltpu.VMEM((G_DEG, CHUNK), jnp.int32),
            pltpu.VMEM((CHUNK, HDIM), jnp.float32),
            pltpu.VMEM((DRAIN, HDIM), jnp.float32),
            pltpu.VMEM_SHARED((ACC2_ROWS, HDIM), jnp.float32),
            pltpu.SemaphoreType.DMA,
        ],
        compiler_params=_SC_PARAMS,
    )
    def k(col_hbm, deg_hbm, colg, ones_v, bounce, acc, ssem):
        core = lax.axis_index("c")
        tile = lax.axis_index("s")
        c0_tile = (core * NS + tile) * per_tile
        n0t = tile * Z_STRIPE

        def ofill(i, carry):
            ones_v[i, pl.ds(0, LANES)] = jnp.ones((LANES,), jnp.float32)
            return carry
        lax.fori_loop(0, CHUNK, ofill, 0)

        def zfill(i, carry):
            bounce[i, pl.ds(0, LANES)] = jnp.zeros((LANES,), jnp.float32)
            return carry
        lax.fori_loop(0, DRAIN, zfill, 0)
        n_b = _n_stripe_chunks(tile, DRAIN)

        def zcopy(i, carry):
            pltpu.sync_copy(bounce, acc.at[pl.ds(n0t + i * DRAIN, DRAIN)])
            return carry
        lax.fori_loop(0, n_b, zcopy, 0)
        plsc.subcore_barrier()

        def group(g, carry):
            c0 = pl.multiple_of(c0_tile + g * G_DEG, G_DEG)
            pltpu.sync_copy(col_hbm.at[pl.ds(c0, G_DEG)], colg)
            sd = [pltpu.async_copy(ones_v, acc.at[colg.at[j]], ssem, add=True)
                  for j in range(G_DEG)]
            for d in sd:
                d.wait()
            return carry

        lax.fori_loop(0, n_groups, group, 0)
        plsc.subcore_barrier()

        def dcopy(i, carry):
            n0 = n0t + i * DRAIN
            pltpu.sync_copy(acc.at[pl.ds(n0, DRAIN)], bounce)
            pltpu.sync_copy(bounce, deg_hbm.at[core, pl.ds(n0, DRAIN)])
            return carry
        lax.fori_loop(0, n_b, dcopy, 0)

    return k(col2)


def _newton_rsqrt(d):
    i = lax.bitcast_convert_type(d, jnp.int32)
    i = jnp.int32(0x5F3759DF) - lax.shift_right_logical(i, 1)
    x = lax.bitcast_convert_type(i, jnp.float32)
    h = d * 0.5
    for _ in range(3):
        x = x * (1.5 - (h * x) * x)
    return jnp.where(d > 0.5, x, 0.0)


def _sc_prep(deg, emb2):
    @functools.partial(
        pl.kernel,
        out_type=(jax.ShapeDtypeStruct((N_NODES, HDIM), jnp.float32),
                  jax.ShapeDtypeStruct((N_NODES, HDIM), jnp.float32),
                  jax.ShapeDtypeStruct((NC, N_NODES, HDIM), jnp.float32)),
        mesh=_MESH,
        scratch_types=[
            pltpu.VMEM((PREP, HDIM), jnp.float32),
            pltpu.VMEM((PREP, HDIM), jnp.float32),
            pltpu.VMEM((PREP, HDIM), jnp.float32),
            pltpu.VMEM((PREP, HDIM), jnp.float32),
            pltpu.SemaphoreType.DMA,
        ],
        compiler_params=_SC_PARAMS,
    )
    def k(deg_hbm, emb_hbm, dinvx_hbm, dinv2x_hbm, t1_hbm,
          d0b, dxb, d2xb, eb, sem):
        core = lax.axis_index("c")
        tile = lax.axis_index("s")
        n0t = tile * Z_STRIPE
        n_c = _n_stripe_chunks(tile, PREP)

        def chunk(i, carry):
            n0 = n0t + i * PREP
            pltpu.sync_copy(deg_hbm.at[0, pl.ds(n0, PREP)], d0b)
            pltpu.sync_copy(deg_hbm.at[1, pl.ds(n0, PREP)], dxb)
            pltpu.sync_copy(emb_hbm.at[core, pl.ds(n0, PREP)], eb)

            def work(j, carry2):
                o = pl.ds(0, LANES)
                d = d0b[j, o] + dxb[j, o]
                dv = _newton_rsqrt(d)
                dxb[j, o] = dv
                d2xb[j, o] = dv * dv
                eb[j, o] = eb[j, o] * dv
                return carry2
            lax.fori_loop(0, PREP, work, 0)

            pltpu.sync_copy(eb, t1_hbm.at[core, pl.ds(n0, PREP)])

            @pl.when(core == 0)
            def _():
                pltpu.sync_copy(dxb, dinvx_hbm.at[pl.ds(n0, PREP)])
                pltpu.sync_copy(d2xb, dinv2x_hbm.at[pl.ds(n0, PREP)])
            return carry

        lax.fori_loop(0, n_c, chunk, 0)

    return k(deg, emb2)


def _sc_layer(t, dinv2x, row2, col2, scaled_out):
    total_chunks = row2.shape[0]
    per_tile = total_chunks // NS
    n_groups = per_tile // G

    out_type = [jax.ShapeDtypeStruct((NC, N_NODES, HDIM), jnp.float32)]
    if scaled_out:
        out_type.append(
            jax.ShapeDtypeStruct((NC, N_NODES, HDIM), jnp.float32))

    @functools.partial(
        pl.kernel,
        out_type=tuple(out_type),
        mesh=_MESH,
        scratch_types=[
            pltpu.VMEM((G, CHUNK), jnp.int32),
            pltpu.VMEM((G, CHUNK), jnp.int32),
            pltpu.VMEM((G, CHUNK, HDIM), jnp.float32),
            pltpu.VMEM((DRAIN, HDIM), jnp.float32),
            pltpu.VMEM((DRAIN, HDIM), jnp.float32),
            pltpu.VMEM_SHARED((ACC2_ROWS, HDIM), jnp.float32),
            pltpu.SemaphoreType.DMA,
            pltpu.SemaphoreType.DMA,
        ],
        compiler_params=_SC_PARAMS,
    )
    def k(t_hbm, d2x_hbm, row_hbm, col_hbm, acc_hbm, *rest):
        if scaled_out:
            (tn_hbm, rowg, colg, bufs, abuf, d2buf, acc, gsem, ssem) = rest
        else:
            (rowg, colg, bufs, abuf, d2buf, acc, gsem, ssem) = rest
        core = lax.axis_index("c")
        tile = lax.axis_index("s")
        c0_tile = tile * per_tile
        n0t = tile * Z_STRIPE

        def zfill(i, carry):
            abuf[i, pl.ds(0, LANES)] = jnp.zeros((LANES,), jnp.float32)
            return carry
        lax.fori_loop(0, DRAIN, zfill, 0)
        n_b = _n_stripe_chunks(tile, DRAIN)

        def zcopy(i, carry):
            pltpu.sync_copy(abuf, acc.at[pl.ds(n0t + i * DRAIN, DRAIN)])
            return carry
        lax.fori_loop(0, n_b, zcopy, 0)
        plsc.subcore_barrier()

        plane = t_hbm.at[core]

        def group(g, carry):
            c0 = pl.multiple_of(c0_tile + g * G, G)
            pltpu.sync_copy(row_hbm.at[pl.ds(c0, G)], rowg)
            pltpu.sync_copy(col_hbm.at[pl.ds(c0, G)], colg)
            gd = [pltpu.async_copy(plane.at[rowg.at[j]], bufs.at[j], gsem)
                  for j in range(G)]
            sd = []
            for j in range(G):
                gd[j].wait()
                sd.append(pltpu.async_copy(bufs.at[j], acc.at[colg.at[j]],
                                           ssem, add=True))
            for d in sd:
                d.wait()
            return carry

        lax.fori_loop(0, n_groups, group, 0)
        plsc.subcore_barrier()

        def dcopy(i, carry):
            n0 = n0t + i * DRAIN
            pltpu.sync_copy(acc.at[pl.ds(n0, DRAIN)], abuf)
            pltpu.sync_copy(abuf, acc_hbm.at[core, pl.ds(n0, DRAIN)])
            if scaled_out:
                pltpu.sync_copy(d2x_hbm.at[pl.ds(n0, DRAIN)], d2buf)

                def scale(j, carry2):
                    o = pl.ds(0, LANES)
                    abuf[j, o] = abuf[j, o] * d2buf[j, o]
                    return carry2
                lax.fori_loop(0, DRAIN, scale, 0)
                pltpu.sync_copy(abuf, tn_hbm.at[core, pl.ds(n0, DRAIN)])
            return carry
        lax.fori_loop(0, n_b, dcopy, 0)

    return k(t, dinv2x, row2, col2)


def _sc_final(emb2, a1, a2, a3, dinvx):
    @functools.partial(
        pl.kernel,
        out_type=jax.ShapeDtypeStruct((NC, N_NODES, HDIM), jnp.float32),
        mesh=_MESH,
        scratch_types=[
            pltpu.VMEM((PREP, HDIM), jnp.float32),
            pltpu.VMEM((PREP, HDIM), jnp.float32),
            pltpu.VMEM((PREP, HDIM), jnp.float32),
            pltpu.VMEM((PREP, HDIM), jnp.float32),
            pltpu.VMEM((PREP, HDIM), jnp.float32),
            pltpu.SemaphoreType.DMA,
        ],
        compiler_params=_SC_PARAMS,
    )
    def k(e_hbm, a1_hbm, a2_hbm, a3_hbm, dx_hbm, out_hbm,
          eb, b1, b2, b3, dvb, sem):
        core = lax.axis_index("c")
        tile = lax.axis_index("s")
        n0t = tile * Z_STRIPE
        n_c = _n_stripe_chunks(tile, PREP)

        def chunk(i, carry):
            n0 = n0t + i * PREP
            pltpu.sync_copy(e_hbm.at[core, pl.ds(n0, PREP)], eb)
            pltpu.sync_copy(a1_hbm.at[core, pl.ds(n0, PREP)], b1)
            pltpu.sync_copy(a2_hbm.at[core, pl.ds(n0, PREP)], b2)
            pltpu.sync_copy(a3_hbm.at[core, pl.ds(n0, PREP)], b3)
            pltpu.sync_copy(dx_hbm.at[pl.ds(n0, PREP)], dvb)

            def mix(j, carry2):
                o = pl.ds(0, LANES)
                s = b1[j, o] + b2[j, o] + b3[j, o]
                eb[j, o] = (eb[j, o] + dvb[j, o] * s) * 0.25
                return carry2
            lax.fori_loop(0, PREP, mix, 0)

            pltpu.sync_copy(eb, out_hbm.at[core, pl.ds(n0, PREP)])
            return carry

        lax.fori_loop(0, n_c, chunk, 0)

    return k(emb2, a1, a2, a3, dinvx)


def kernel(edge_index, user_emb, item_emb):
    row = edge_index[0].astype(jnp.int32)
    col = edge_index[1].astype(jnp.int32)

    n_edges = row.shape[0]
    step = NS * CHUNK * G * NC
    e_pad = ((n_edges + step - 1) // step) * step
    pad = e_pad - n_edges
    row2 = jnp.concatenate([row, jnp.zeros((pad,), jnp.int32)])
    col2 = jnp.concatenate([col, jnp.full((pad,), N_NODES, jnp.int32)])
    row2 = row2.reshape(-1, CHUNK)
    col2 = col2.reshape(-1, CHUNK)

    emb2 = jnp.stack([
        jnp.concatenate([user_emb[:, :HDIM], item_emb[:, :HDIM]], axis=0),
        jnp.concatenate([user_emb[:, HDIM:], item_emb[:, HDIM:]], axis=0),
    ])

    deg = _sc_degree(col2)
    dinvx, dinv2x, t = _sc_prep(deg, emb2)

    a1, t = _sc_layer(t, dinv2x, row2, col2, scaled_out=True)
    a2, t = _sc_layer(t, dinv2x, row2, col2, scaled_out=True)
    (a3,) = _sc_layer(t, dinv2x, row2, col2, scaled_out=False)

    final2 = _sc_final(emb2, a1, a2, a3, dinvx)
    final = jnp.concatenate([final2[0], final2[1]], axis=1)
    return (final[:N_USERS], final[N_USERS:])
